# baseline (speedup 1.0000x reference)
import functools

import jax
import jax.numpy as jnp
from jax import lax
from jax.experimental import pallas as pl
from jax.experimental.pallas import tpu as pltpu
from jax.experimental.pallas import tpu_sc as plsc

B = 4096
N_TOK = 200
D = 64
NC = 2
NS = 16
NW = NC * NS
BL = 128
NBC = B // BL
E8 = D // 8

_mesh = plsc.VectorSubcoreMesh(core_axis_name="c", subcore_axis_name="s")


@functools.partial(
    pl.kernel,
    mesh=_mesh,
    compiler_params=pltpu.CompilerParams(
        use_tc_tiling_on_sc=False, needs_layout_passes=False
    ),
    out_type=jax.ShapeDtypeStruct((N_TOK, E8, NBC, 8, BL), jnp.float32),
    scratch_types=[
        pltpu.VMEM((1, BL), jnp.int32),
        pltpu.VMEM((BL, D), jnp.float32),
        pltpu.VMEM((E8, 1, 8, BL), jnp.float32),
        pltpu.VMEM((N_TOK, D), jnp.float32),
        pltpu.SemaphoreType.DMA,
    ],
)
def _emb_kernel(xp_hbm, table_hbm, pos_hbm, out_hbm, idx_v, rows_v, stage_v,
                pos_v, sem):
    bc = lax.axis_index("s") * NC + lax.axis_index("c")
    pltpu.sync_copy(pos_hbm, pos_v)
    viota = lax.iota(jnp.int32, 16)

    def token_body(t, carry):
        tt = t // 8
        ts = t % 8
        pltpu.sync_copy(xp_hbm.at[tt, bc, ts], idx_v.at[0])
        pltpu.async_copy(table_hbm.at[idx_v.at[0]], rows_v, sem).wait()

        tvec = jnp.full((16,), t, dtype=jnp.int32)

        def e8_body(e8, c):
            for es in range(8):
                e = e8 * 8 + es
                cvec = jnp.full((16,), e, dtype=jnp.int32)
                p = plsc.load_gather(pos_v, [tvec, cvec])
                for bl0 in range(8):
                    rvec = viota + (bl0 * 16)
                    vals = plsc.load_gather(rows_v, [rvec, cvec])
                    stage_v[e8, 0, es, pl.ds(bl0 * 16, 16)] = vals + p
            return c

        lax.fori_loop(0, E8, e8_body, 0)
        pltpu.sync_copy(stage_v, out_hbm.at[t, :, pl.ds(bc, 1)])
        return carry

    lax.fori_loop(0, N_TOK, token_body, 0)


def kernel(x, table, pos_embedding):
    xp = (
        x.astype(jnp.int32)
        .T.reshape(N_TOK // 8, 8, NBC, BL)
        .transpose((0, 2, 1, 3))
    )
    y = _emb_kernel(xp, table, pos_embedding)
    return y.transpose((2, 4, 0, 1, 3)).reshape(B, N_TOK, D)


# speedup vs baseline: 1.3732x; 1.3732x over previous
import functools

import jax
import jax.numpy as jnp
from jax import lax
from jax.experimental import pallas as pl
from jax.experimental.pallas import tpu as pltpu
from jax.experimental.pallas import tpu_sc as plsc

B = 4096
N_TOK = 200
D = 64
NC = 2
NS = 16
NW = NC * NS
BL = 128
NBC = B // BL
E8 = D // 8

_mesh = plsc.VectorSubcoreMesh(core_axis_name="c", subcore_axis_name="s")


@functools.partial(
    pl.kernel,
    mesh=_mesh,
    compiler_params=pltpu.CompilerParams(
        use_tc_tiling_on_sc=False, needs_layout_passes=False
    ),
    out_type=jax.ShapeDtypeStruct((N_TOK, E8, NBC, 8, BL), jnp.float32),
    scratch_types=[
        pltpu.VMEM((2, BL), jnp.int32),
        pltpu.VMEM((2, BL, D), jnp.float32),
        pltpu.VMEM((2, E8, 1, 8, BL), jnp.float32),
        pltpu.VMEM((N_TOK * D,), jnp.float32),
        pltpu.SemaphoreType.DMA((2,)),
        pltpu.SemaphoreType.DMA((2,)),
    ],
)
def _emb_kernel(xp_hbm, table_hbm, pos_hbm, out_hbm, idx_v, rows_v, stage_v,
                pos_v, gsem, wsem):
    bc = lax.axis_index("s") * NC + lax.axis_index("c")
    pltpu.sync_copy(pos_hbm, pos_v)
    viota = lax.iota(jnp.int32, 16)

    pltpu.sync_copy(xp_hbm.at[0, bc, 0], idx_v.at[0])
    pltpu.async_copy(table_hbm.at[idx_v.at[0]], rows_v.at[0], gsem.at[0])

    def token_body(t, carry):
        buf = lax.rem(t, 2)
        nbuf = 1 - buf
        tn = t + 1

        @pl.when(tn < N_TOK)
        def _():
            pltpu.sync_copy(xp_hbm.at[tn // 8, bc, tn % 8], idx_v.at[nbuf])
            pltpu.async_copy(
                table_hbm.at[idx_v.at[nbuf]], rows_v.at[nbuf], gsem.at[nbuf]
            )

        pltpu.make_async_copy(
            table_hbm.at[idx_v.at[buf]], rows_v.at[buf], gsem.at[buf]
        ).wait()

        @pl.when(t >= 2)
        def _():
            pltpu.make_async_copy(
                stage_v.at[buf], out_hbm.at[t, :, pl.ds(bc, 1)], wsem.at[buf]
            ).wait()

        tbase = t * D

        @plsc.parallel_loop(0, E8, unroll=2)
        def e8_body(e8):
            for es in range(8):
                e = e8 * 8 + es
                cvec = jnp.full((16,), e, dtype=jnp.int32)
                p = plsc.load_gather(
                    pos_v, [jnp.full((16,), tbase + e, dtype=jnp.int32)]
                )
                for bl0 in range(8):
                    rvec = viota + (bl0 * 16)
                    vals = plsc.load_gather(rows_v.at[buf], [rvec, cvec])
                    stage_v[buf, e8, 0, es, pl.ds(bl0 * 16, 16)] = vals + p

        pltpu.async_copy(
            stage_v.at[buf], out_hbm.at[t, :, pl.ds(bc, 1)], wsem.at[buf]
        )
        return carry

    lax.fori_loop(0, N_TOK, token_body, 0)

    for k in range(2):
        pltpu.make_async_copy(
            stage_v.at[k],
            out_hbm.at[N_TOK - 2 + k, :, pl.ds(bc, 1)],
            wsem.at[k],
        ).wait()


def kernel(x, table, pos_embedding):
    xp = (
        x.astype(jnp.int32)
        .T.reshape(N_TOK // 8, 8, NBC, BL)
        .transpose((0, 2, 1, 3))
    )
    y = _emb_kernel(xp, table, pos_embedding.reshape(-1))
    return y.transpose((2, 4, 0, 1, 3)).reshape(B, N_TOK, D)


# speedup vs baseline: 1.8239x; 1.3283x over previous
import functools

import jax
import jax.numpy as jnp
from jax import lax
from jax.experimental import pallas as pl
from jax.experimental.pallas import tpu as pltpu
from jax.experimental.pallas import tpu_sc as plsc

B = 4096
N_TOK = 200
D = 64
NC = 2
NS = 16
NW = NC * NS
BL = 128
NBC = B // BL
E8 = D // 8

_mesh = plsc.VectorSubcoreMesh(core_axis_name="c", subcore_axis_name="s")


@functools.partial(
    pl.kernel,
    mesh=_mesh,
    compiler_params=pltpu.CompilerParams(
        use_tc_tiling_on_sc=False, needs_layout_passes=False
    ),
    out_type=jax.ShapeDtypeStruct((N_TOK, E8, NBC, 8, BL), jnp.float32),
    scratch_types=[
        pltpu.VMEM((2, BL), jnp.int32),
        pltpu.VMEM((2, BL, D), jnp.float32),
        pltpu.VMEM((2, E8, 1, 8, BL), jnp.float32),
        pltpu.VMEM((N_TOK * D,), jnp.float32),
        pltpu.SemaphoreType.DMA((2,)),
        pltpu.SemaphoreType.DMA((2,)),
    ],
)
def _emb_kernel(xp_hbm, table_hbm, pos_hbm, out_hbm, idx_v, rows_v, stage_v,
                pos_v, gsem, wsem):
    bc = lax.axis_index("s") * NC + lax.axis_index("c")
    pltpu.sync_copy(pos_hbm, pos_v)
    viota = lax.iota(jnp.int32, 16)

    pltpu.sync_copy(xp_hbm.at[0, bc, 0], idx_v.at[0])
    pltpu.async_copy(table_hbm.at[idx_v.at[0]], rows_v.at[0], gsem.at[0])

    def token_body(t, carry):
        buf = lax.rem(t, 2)
        nbuf = 1 - buf
        tn = t + 1

        @pl.when(tn < N_TOK)
        def _():
            pltpu.sync_copy(xp_hbm.at[tn // 8, bc, tn % 8], idx_v.at[nbuf])
            pltpu.async_copy(
                table_hbm.at[idx_v.at[nbuf]], rows_v.at[nbuf], gsem.at[nbuf]
            )

        pltpu.make_async_copy(
            table_hbm.at[idx_v.at[buf]], rows_v.at[buf], gsem.at[buf]
        ).wait()

        @pl.when(t >= 2)
        def _():
            pltpu.make_async_copy(
                stage_v.at[buf], out_hbm.at[t, :, pl.ds(bc, 1)], wsem.at[buf]
            ).wait()

        tbase = t * D
        bufvec = jnp.full((16,), buf, dtype=jnp.int32)
        zvec = jnp.zeros((16,), dtype=jnp.int32)

        @plsc.parallel_loop(0, D, 16)
        def _e0_body(e0):
            for k in range(16):
                rot = (viota + k) & 15
                cvec = rot + e0
                e8vec = cvec >> 3
                esvec = cvec & 7
                p = plsc.load_gather(pos_v, [rot + (tbase + e0)])
                for r0 in range(0, BL, 16):
                    rvec = viota + r0
                    vals = plsc.load_gather(rows_v.at[buf], [rvec, cvec])
                    plsc.store_scatter(
                        stage_v,
                        [bufvec, e8vec, zvec, esvec, rvec],
                        vals + p,
                    )

        pltpu.async_copy(
            stage_v.at[buf], out_hbm.at[t, :, pl.ds(bc, 1)], wsem.at[buf]
        )
        return carry

    lax.fori_loop(0, N_TOK, token_body, 0)

    for k in range(2):
        pltpu.make_async_copy(
            stage_v.at[k],
            out_hbm.at[N_TOK - 2 + k, :, pl.ds(bc, 1)],
            wsem.at[k],
        ).wait()


def kernel(x, table, pos_embedding):
    xp = (
        x.astype(jnp.int32)
        .T.reshape(N_TOK // 8, 8, NBC, BL)
        .transpose((0, 2, 1, 3))
    )
    y = _emb_kernel(xp, table, pos_embedding.reshape(-1))
    return y.transpose((2, 4, 0, 1, 3)).reshape(B, N_TOK, D)


# speedup vs baseline: 2.7599x; 1.5131x over previous
import functools

import jax
import jax.numpy as jnp
from jax import lax
from jax.experimental import pallas as pl
from jax.experimental.pallas import tpu as pltpu
from jax.experimental.pallas import tpu_sc as plsc

B = 4096
N_TOK = 200
D = 64
NC = 2
NS = 16
NW = NC * NS
BL = 128
NBC = B // BL
E8 = D // 8

_mesh = plsc.VectorSubcoreMesh(core_axis_name="c", subcore_axis_name="s")


@functools.partial(
    pl.kernel,
    mesh=_mesh,
    compiler_params=pltpu.CompilerParams(
        use_tc_tiling_on_sc=False, needs_layout_passes=False
    ),
    out_type=jax.ShapeDtypeStruct((N_TOK, E8, NBC, 8, BL), jnp.float32),
    scratch_types=[
        pltpu.VMEM((2, BL), jnp.int32),
        pltpu.VMEM((2, BL, D), jnp.float32),
        pltpu.VMEM((2, E8, 1, 8, BL), jnp.float32),
        pltpu.VMEM((N_TOK * D,), jnp.float32),
        pltpu.SemaphoreType.DMA((2,)),
        pltpu.SemaphoreType.DMA((2,)),
        pltpu.SemaphoreType.DMA((2,)),
    ],
)
def _emb_kernel(xp_hbm, table_hbm, pos_hbm, out_hbm, idx_v, rows_v, stage_v,
                pos_v, isem, gsem, wsem):
    bc = lax.axis_index("s") * NC + lax.axis_index("c")
    pltpu.sync_copy(pos_hbm, pos_v)
    viota = lax.iota(jnp.int32, 16)

    pltpu.sync_copy(xp_hbm.at[0, bc, 0], idx_v.at[0])
    pltpu.async_copy(table_hbm.at[idx_v.at[0]], rows_v.at[0], gsem.at[0])
    pltpu.async_copy(xp_hbm.at[0, bc, 1], idx_v.at[1], isem.at[1])

    def token_body(t, carry):
        buf = lax.rem(t, 2)
        nbuf = 1 - buf
        tn = t + 1
        tp = t + 2

        pltpu.make_async_copy(
            table_hbm.at[idx_v.at[buf]], rows_v.at[buf], gsem.at[buf]
        ).wait()

        @pl.when(tn < N_TOK)
        def _():
            pltpu.make_async_copy(
                xp_hbm.at[tn // 8, bc, tn % 8], idx_v.at[nbuf], isem.at[nbuf]
            ).wait()
            pltpu.async_copy(
                table_hbm.at[idx_v.at[nbuf]], rows_v.at[nbuf], gsem.at[nbuf]
            )

        @pl.when(tp < N_TOK)
        def _():
            pltpu.async_copy(
                xp_hbm.at[tp // 8, bc, tp % 8], idx_v.at[buf], isem.at[buf]
            )

        @pl.when(t >= 2)
        def _():
            pltpu.make_async_copy(
                stage_v.at[buf], out_hbm.at[t, :, pl.ds(bc, 1)], wsem.at[buf]
            ).wait()

        tbase = t * D
        bufvec = jnp.full((16,), buf, dtype=jnp.int32)
        zvec = jnp.zeros((16,), dtype=jnp.int32)

        @plsc.parallel_loop(0, D, 1, unroll=2)
        def _ek_body(i):
            rot = (viota + i) & 15
            e0 = i & 48
            cvec = rot + e0
            e8vec = cvec >> 3
            esvec = cvec & 7
            p = plsc.load_gather(pos_v, [rot + (tbase + e0)])
            for r0 in range(0, BL, 16):
                rvec = viota + r0
                vals = plsc.load_gather(rows_v.at[buf], [rvec, cvec])
                plsc.store_scatter(
                    stage_v,
                    [bufvec, e8vec, zvec, esvec, rvec],
                    vals + p,
                )

        pltpu.async_copy(
            stage_v.at[buf], out_hbm.at[t, :, pl.ds(bc, 1)], wsem.at[buf]
        )
        return carry

    lax.fori_loop(0, N_TOK, token_body, 0)

    for k in range(2):
        pltpu.make_async_copy(
            stage_v.at[k],
            out_hbm.at[N_TOK - 2 + k, :, pl.ds(bc, 1)],
            wsem.at[k],
        ).wait()


def kernel(x, table, pos_embedding):
    xp = (
        x.astype(jnp.int32)
        .T.reshape(N_TOK // 8, 8, NBC, BL)
        .transpose((0, 2, 1, 3))
    )
    y = _emb_kernel(xp, table, pos_embedding.reshape(-1))
    return y.transpose((2, 4, 0, 1, 3)).reshape(B, N_TOK, D)


# speedup vs baseline: 2.9878x; 1.0826x over previous
import functools

import jax
import jax.numpy as jnp
from jax import lax
from jax.experimental import pallas as pl
from jax.experimental.pallas import tpu as pltpu
from jax.experimental.pallas import tpu_sc as plsc

B = 4096
N_TOK = 200
D = 64
NC = 2
NS = 16
NW = NC * NS
BL = 128
NBC = B // BL
E8 = D // 8
TP = 2
NP = N_TOK // TP

_mesh = plsc.VectorSubcoreMesh(core_axis_name="c", subcore_axis_name="s")


@functools.partial(
    pl.kernel,
    mesh=_mesh,
    compiler_params=pltpu.CompilerParams(
        use_tc_tiling_on_sc=False, needs_layout_passes=False
    ),
    out_type=jax.ShapeDtypeStruct((N_TOK, E8, NBC, 8, BL), jnp.float32),
    scratch_types=[
        pltpu.VMEM((2, TP, BL), jnp.int32),
        pltpu.VMEM((2, TP, BL, D), jnp.float32),
        pltpu.VMEM((2, TP, E8, 1, 8, BL), jnp.float32),
        pltpu.VMEM((N_TOK * D,), jnp.float32),
        pltpu.SemaphoreType.DMA((2,)),
        pltpu.SemaphoreType.DMA((2,)),
        pltpu.SemaphoreType.DMA((2,)),
    ],
)
def _emb_kernel(xp_hbm, table_hbm, pos_hbm, out_hbm, idx_v, rows_v, stage_v,
                pos_v, isem, gsem, wsem):
    bc = lax.axis_index("s") * NC + lax.axis_index("c")
    pltpu.sync_copy(pos_hbm, pos_v)
    viota = lax.iota(jnp.int32, 16)

    def idx_src(p):
        return xp_hbm.at[p // 4, bc, pl.ds((p % 4) * 2, TP)]

    pltpu.sync_copy(idx_src(0), idx_v.at[0])
    for j in range(TP):
        pltpu.async_copy(
            table_hbm.at[idx_v.at[0, j]], rows_v.at[0, j], gsem.at[0]
        )
    pltpu.async_copy(idx_src(1), idx_v.at[1], isem.at[1])

    def pair_body(p, carry):
        buf = lax.rem(p, 2)
        nbuf = 1 - buf
        t = p * TP

        for j in range(TP):
            pltpu.make_async_copy(
                table_hbm.at[idx_v.at[buf, j]], rows_v.at[buf, j],
                gsem.at[buf],
            ).wait()

        @pl.when(p + 1 < NP)
        def _():
            pltpu.make_async_copy(
                idx_src(p + 1), idx_v.at[nbuf], isem.at[nbuf]
            ).wait()
            for j in range(TP):
                pltpu.async_copy(
                    table_hbm.at[idx_v.at[nbuf, j]], rows_v.at[nbuf, j],
                    gsem.at[nbuf],
                )

        @pl.when(p + 2 < NP)
        def _():
            pltpu.async_copy(idx_src(p + 2), idx_v.at[buf], isem.at[buf])

        @pl.when(p >= 2)
        def _():
            pltpu.make_async_copy(
                stage_v.at[buf],
                out_hbm.at[pl.ds(t, TP), :, pl.ds(bc, 1)],
                wsem.at[buf],
            ).wait()

        @plsc.parallel_loop(0, TP * D, 1, unroll=2)
        def _ek_body(i):
            j = i >> 6
            ii = i & 63
            rot = (viota + ii) & 15
            e0 = ii & 48
            cvec = rot + e0
            e8vec = cvec >> 3
            esvec = cvec & 7
            p_vec = plsc.load_gather(pos_v, [rot + ((t + j) * D + e0)])
            rows_j = rows_v.at[buf, j]
            stage_j = stage_v.at[buf, j]
            for r0 in range(0, BL, 16):
                rvec = viota + r0
                vals = plsc.load_gather(rows_j, [rvec, cvec])
                plsc.store_scatter(
                    stage_j, [e8vec, jnp.zeros((16,), jnp.int32), esvec, rvec],
                    vals + p_vec,
                )

        pltpu.async_copy(
            stage_v.at[buf],
            out_hbm.at[pl.ds(t, TP), :, pl.ds(bc, 1)],
            wsem.at[buf],
        )
        return carry

    lax.fori_loop(0, NP, pair_body, 0)

    for k in range(2):
        pltpu.make_async_copy(
            stage_v.at[k],
            out_hbm.at[pl.ds((NP - 2 + k) * TP, TP), :, pl.ds(bc, 1)],
            wsem.at[k],
        ).wait()


def kernel(x, table, pos_embedding):
    xp = (
        x.astype(jnp.int32)
        .T.reshape(N_TOK // 8, 8, NBC, BL)
        .transpose((0, 2, 1, 3))
    )
    y = _emb_kernel(xp, table, pos_embedding.reshape(-1))
    return y.transpose((2, 4, 0, 1, 3)).reshape(B, N_TOK, D)
